# baseline (speedup 1.0000x reference)
import functools

import jax
import jax.numpy as jnp
from jax import lax
from jax.experimental import pallas as pl
from jax.experimental.pallas import tpu as pltpu
from jax.experimental.pallas import tpu_sc as plsc

_BATCH = 16384
_D = 32
_ROWS = 1000000
_NC = 2
_NS = 16
_NW = _NC * _NS
_BPW = _BATCH // _NW
_CHUNK = 128
_NCHUNK = _BPW // _CHUNK
_L = 16
_GPC = _CHUNK // _L
_TBLK = 8192


def _tp_body(in_ref, eye_ref, o_ref):
    o_ref[...] = lax.dot_general(
        in_ref[...], eye_ref[...], (((0,), (0,)), ((), ())),
        preferred_element_type=jnp.float32)


def _tc_transpose(tab_t, eye):
    grid = pl.cdiv(_ROWS, _TBLK)
    return pl.pallas_call(
        _tp_body,
        grid=(grid,),
        in_specs=[pl.BlockSpec((_D, _TBLK), lambda i: (0, i)),
                  pl.BlockSpec((_D, _D), lambda i: (0, 0))],
        out_specs=pl.BlockSpec((_TBLK, _D), lambda i: (i, 0)),
        out_shape=jax.ShapeDtypeStruct((_ROWS, _D), jnp.float32),
    )(tab_t, eye)


def _sc_gather_mul(uidx_hbm, iidx_hbm, utab_hbm, itab_hbm, out_hbm,
                   uidx_v, iidx_v, usup_i, isup_i, usup, vsup, x_v,
                   sem_u0, sem_u1, sem_i0, sem_i1):
    wid = lax.axis_index("s") * _NC + lax.axis_index("c")
    base = wid * _BPW
    pltpu.sync_copy(uidx_hbm.at[pl.ds(base, _BPW)], uidx_v)
    pltpu.sync_copy(iidx_hbm.at[pl.ds(base, _BPW)], iidx_v)

    for j in range(_NCHUNK):
        for g in range(_GPC):
            s = pl.ds(j * _CHUNK + g * _L, _L)
            d = pl.ds(g * _L, _L)
            usup_i[j, d] = lax.shift_right_logical(uidx_v[s], 2)
            isup_i[j, d] = lax.shift_right_logical(iidx_v[s], 2)

    sem_u = (sem_u0, sem_u1)
    sem_i = (sem_i0, sem_i1)

    def fire(j):
        b = j % 2
        cu = pltpu.async_copy(utab_hbm.at[usup_i.at[j]], usup.at[b], sem_u[b])
        ci = pltpu.async_copy(itab_hbm.at[isup_i.at[j]], vsup.at[b], sem_i[b])
        return cu, ci

    lane = lax.iota(jnp.int32, _L)
    pending = fire(0)

    for j in range(_NCHUNK):
        nxt = fire(j + 1) if j + 1 < _NCHUNK else None
        pending[0].wait()
        pending[1].wait()
        b = j % 2
        usup_b = usup.at[b]
        vsup_b = vsup.at[b]

        def group(g, carry, j=j, usup_b=usup_b, vsup_b=vsup_b):
            s = pl.ds(j * _CHUNK + g * _L, _L)
            uix = uidx_v[s]
            iix = iidx_v[s]
            rloc = g * _L + lane
            ucol = (uix & 3) * _D
            vcol = (iix & 3) * _D
            obase = (j * _CHUNK) * _D + rloc * _D
            for d in range(_D):
                uvec = plsc.load_gather(usup_b, (rloc, ucol + d))
                vvec = plsc.load_gather(vsup_b, (rloc, vcol + d))
                plsc.store_scatter(x_v, (obase + d,), uvec * vvec)
            return carry

        lax.fori_loop(0, _GPC, group, 0)
        pending = nxt

    pltpu.sync_copy(x_v, out_hbm.at[pl.ds(base * _D, _BPW * _D)])


def _sc_call(user_idx, item_idx, utab_rm, itab_rm):
    mesh = plsc.VectorSubcoreMesh(
        core_axis_name="c", subcore_axis_name="s",
        num_cores=_NC, num_subcores=_NS)
    fn = functools.partial(
        pl.kernel,
        mesh=mesh,
        out_type=jax.ShapeDtypeStruct((_BATCH * _D,), jnp.float32),
        scratch_types=[
            pltpu.VMEM((_BPW,), jnp.int32),
            pltpu.VMEM((_BPW,), jnp.int32),
            pltpu.VMEM((_NCHUNK, _CHUNK), jnp.int32),
            pltpu.VMEM((_NCHUNK, _CHUNK), jnp.int32),
            pltpu.VMEM((2, _CHUNK, 128), jnp.float32),
            pltpu.VMEM((2, _CHUNK, 128), jnp.float32),
            pltpu.VMEM((_BPW * _D,), jnp.float32),
            pltpu.SemaphoreType.DMA,
            pltpu.SemaphoreType.DMA,
            pltpu.SemaphoreType.DMA,
            pltpu.SemaphoreType.DMA,
        ],
        compiler_params=pltpu.CompilerParams(needs_layout_passes=False),
    )(_sc_gather_mul)
    uidx = user_idx.astype(jnp.int32)
    iidx = item_idx.astype(jnp.int32)
    utab = utab_rm.reshape(-1, 128)
    itab = itab_rm.reshape(-1, 128)
    return fn(uidx, iidx, utab, itab)


def _mlp_body(x_ref, w1_ref, b1_ref, w2_ref, b2_ref, o_ref):
    x = x_ref[...]
    h = jnp.dot(x, w1_ref[...], preferred_element_type=jnp.float32)
    h = jnp.maximum(h + b1_ref[...], 0.0)
    z = jnp.dot(h, w2_ref[...], preferred_element_type=jnp.float32)
    z = z + b2_ref[...]
    o_ref[...] = 1.0 / (1.0 + jnp.exp(-z))


def _tc_mlp(x, W1, b1, W2, b2):
    out = pl.pallas_call(
        _mlp_body,
        out_shape=jax.ShapeDtypeStruct((_BATCH, 1), jnp.float32),
    )(x, W1, b1.reshape(1, 16), W2, b2.reshape(1, 1))
    return out.reshape(_BATCH)


@jax.jit
def kernel(user_idx, item_idx, user_table, item_table, W1, b1, W2, b2):
    eye = jnp.eye(_D, dtype=jnp.float32)
    utab_rm = _tc_transpose(user_table.T, eye)
    itab_rm = _tc_transpose(item_table.T, eye)
    x = _sc_call(user_idx, item_idx, utab_rm, itab_rm)
    return _tc_mlp(x.reshape(_BATCH, _D), W1, b1, W2, b2)


# speedup vs baseline: 1.3327x; 1.3327x over previous
import functools

import jax
import jax.numpy as jnp
from jax import lax
from jax.experimental import pallas as pl
from jax.experimental.pallas import tpu as pltpu
from jax.experimental.pallas import tpu_sc as plsc

_BATCH = 16384
_D = 32
_NC = 2
_NS = 16
_NW = _NC * _NS
_BPW = _BATCH // _NW
_CHUNK = 128
_NCHUNK = _BPW // _CHUNK
_L = 16


def _sc_gather_mul(uidx_hbm, iidx_hbm, utab_hbm, itab_hbm, out_hbm,
                   uidx_v, iidx_v, u_v, v_v, sem_u, sem_i):
    wid = lax.axis_index("s") * _NC + lax.axis_index("c")
    crow = wid * _NCHUNK
    pltpu.sync_copy(uidx_hbm.at[pl.ds(crow, _NCHUNK)], uidx_v)
    pltpu.sync_copy(iidx_hbm.at[pl.ds(crow, _NCHUNK)], iidx_v)

    copies = []
    for j in range(_NCHUNK):
        dst = u_v.at[pl.ds(j * _CHUNK, _CHUNK)]
        copies.append(pltpu.async_copy(utab_hbm.at[uidx_v.at[j]], dst, sem_u))
    for j in range(_NCHUNK):
        dst = v_v.at[pl.ds(j * _CHUNK, _CHUNK)]
        copies.append(pltpu.async_copy(itab_hbm.at[iidx_v.at[j]], dst, sem_i))
    for c in copies:
        c.wait()

    def body(r, carry):
        a = u_v[r, pl.ds(0, _L)] * v_v[r, pl.ds(0, _L)]
        u_v[r, pl.ds(0, _L)] = a
        b = u_v[r, pl.ds(_L, _L)] * v_v[r, pl.ds(_L, _L)]
        u_v[r, pl.ds(_L, _L)] = b
        return carry

    lax.fori_loop(0, _BPW, body, 0, unroll=4)

    pltpu.sync_copy(u_v, out_hbm.at[pl.ds(wid * _BPW, _BPW)])


@jax.jit
def _sc_call(user_idx, item_idx, user_table, item_table):
    mesh = plsc.VectorSubcoreMesh(
        core_axis_name="c", subcore_axis_name="s",
        num_cores=_NC, num_subcores=_NS)
    fn = functools.partial(
        pl.kernel,
        mesh=mesh,
        out_type=jax.ShapeDtypeStruct((_BATCH, _D), jnp.float32),
        scratch_types=[
            pltpu.VMEM((_NCHUNK, _CHUNK), jnp.int32),
            pltpu.VMEM((_NCHUNK, _CHUNK), jnp.int32),
            pltpu.VMEM((_BPW, _D), jnp.float32),
            pltpu.VMEM((_BPW, _D), jnp.float32),
            pltpu.SemaphoreType.DMA,
            pltpu.SemaphoreType.DMA,
        ],
        compiler_params=pltpu.CompilerParams(use_tc_tiling_on_sc=False),
    )(_sc_gather_mul)
    uidx = user_idx.reshape(_NW * _NCHUNK, _CHUNK).astype(jnp.int32)
    iidx = item_idx.reshape(_NW * _NCHUNK, _CHUNK).astype(jnp.int32)
    return fn(uidx, iidx, user_table, item_table)


def _mlp_body(x_ref, w1_ref, b1_ref, w2_ref, b2_ref, o_ref):
    x = x_ref[...]
    h = jnp.dot(x, w1_ref[...], preferred_element_type=jnp.float32)
    h = jnp.maximum(h + b1_ref[...], 0.0)
    z = jnp.dot(h, w2_ref[...], preferred_element_type=jnp.float32)
    z = z + b2_ref[...]
    o_ref[...] = 1.0 / (1.0 + jnp.exp(-z))


@jax.jit
def _tc_mlp(x, W1, b1, W2, b2):
    out = pl.pallas_call(
        _mlp_body,
        out_shape=jax.ShapeDtypeStruct((_BATCH, 1), jnp.float32),
    )(x, W1, b1.reshape(1, 16), W2, b2.reshape(1, 1))
    return out.reshape(_BATCH)


def kernel(user_idx, item_idx, user_table, item_table, W1, b1, W2, b2):
    x = _sc_call(user_idx, item_idx, user_table, item_table)
    return _tc_mlp(x, W1, b1, W2, b2)
